# Initial kernel scaffold; baseline (speedup 1.0000x reference)
import jax
import jax.numpy as jnp
from jax.experimental import pallas as pl


def kernel(x, edge_index, edge_weight, res_W, res_b, res_ln_w, res_ln_b, conv_W, conv_b, ln_w, ln_b):
    raise NotImplementedError("write your pallas kernel here")


# speedup vs baseline: 5.6457x; 5.6457x over previous
import functools
import math

import jax
import jax.numpy as jnp
from jax import lax
from jax.experimental import pallas as pl
from jax.experimental.pallas import tpu as pltpu
from jax.experimental.pallas import tpu_sc as plsc

_N = 10000
_D = 128
_E = 320000
_CONVS = 10

_NC = 2
_NS = 16
_NW = _NC * _NS

_CHUNK = 128
_EPT = -(-_E // _NW)
_NCHUNK = -(-_EPT // _CHUNK)
_EPT_PAD = _NCHUNK * _CHUNK
_NPAD = 10240
_RPS = _NPAD // _NS

_BLK = 2000
_GRID = _N // _BLK


def _f32(x):
    return jnp.asarray(x, jnp.float32)


def _sc_mesh():
    return plsc.VectorSubcoreMesh(core_axis_name="c", subcore_axis_name="s")


_SC_PARAMS = pltpu.CompilerParams(needs_layout_passes=False)


def _deg_body(dst_hbm, ew_hbm, out_hbm, dstv, ewv, zbuf, acc):
    cid = lax.axis_index("c")
    sid = lax.axis_index("s")
    wid = sid * _NC + cid

    pltpu.sync_copy(dst_hbm.at[wid], dstv)
    pltpu.sync_copy(ew_hbm.at[wid], ewv)

    def _z(i, carry):
        zbuf[pl.ds(16 * i, 16)] = jnp.zeros((16,), jnp.float32)
        return carry

    lax.fori_loop(0, _RPS // 16, _z, 0)
    pltpu.sync_copy(zbuf, acc.at[pl.ds(sid * _RPS, _RPS)])
    plsc.subcore_barrier()

    def _chunk(k, carry):
        pltpu.sync_copy(ewv.at[k], acc.at[dstv.at[k]], add=True)
        return carry

    lax.fori_loop(0, _NCHUNK, _chunk, 0)
    plsc.subcore_barrier()

    sl = pl.ds(sid * _RPS, _RPS)
    pltpu.sync_copy(acc.at[sl], out_hbm.at[cid].at[sl])


def _deg_call(dstp, ewp):
    kfn = pl.kernel(
        _deg_body,
        out_type=jax.ShapeDtypeStruct((_NC, _NPAD), jnp.float32),
        mesh=_sc_mesh(),
        scratch_types=[
            pltpu.VMEM((_NCHUNK, _CHUNK), jnp.int32),
            pltpu.VMEM((_NCHUNK, _CHUNK), jnp.float32),
            pltpu.VMEM((_RPS,), jnp.float32),
            pltpu.VMEM_SHARED((_NPAD,), jnp.float32),
        ],
        compiler_params=_SC_PARAMS,
        name="sc_deg",
    )
    return kfn(dstp, ewp)


def _agg_body(y_hbm, src_hbm, dst_hbm, ew_hbm, out_hbm, srcv, dstv, ewv, buf, acc, gsem):
    cid = lax.axis_index("c")
    sid = lax.axis_index("s")
    wid = sid * _NC + cid

    pltpu.sync_copy(src_hbm.at[wid], srcv)
    pltpu.sync_copy(dst_hbm.at[wid], dstv)
    pltpu.sync_copy(ew_hbm.at[wid], ewv)

    def _zrow(r, carry):
        for j in range(8):
            buf[r, pl.ds(16 * j, 16)] = jnp.zeros((16,), jnp.float32)
        return carry

    lax.fori_loop(0, _CHUNK, _zrow, 0)
    for t in range(_RPS // _CHUNK):
        pltpu.sync_copy(
            buf,
            acc.at[pl.ds(sid * _RPS + t * _CHUNK, _CHUNK)],
        )
    plsc.subcore_barrier()

    def _chunk(k, carry):
        pltpu.async_copy(y_hbm.at[srcv.at[k]], buf, gsem).wait()

        def _row(r, carry2):
            s = plsc.load_gather(
                ewv, [jnp.full((16,), k * _CHUNK + r, jnp.int32)]
            )
            for j in range(8):
                buf[r, pl.ds(16 * j, 16)] = buf[r, pl.ds(16 * j, 16)] * s
            return carry2

        lax.fori_loop(0, _CHUNK, _row, 0)
        pltpu.sync_copy(buf, acc.at[dstv.at[k]], add=True)
        return carry

    lax.fori_loop(0, _NCHUNK, _chunk, 0)
    plsc.subcore_barrier()

    for t in range(_RPS // _CHUNK):
        sl = pl.ds(sid * _RPS + t * _CHUNK, _CHUNK)
        pltpu.sync_copy(acc.at[sl], out_hbm.at[cid].at[sl])


def _agg_call(y, srcp, dstp, ewp):
    kfn = pl.kernel(
        _agg_body,
        out_type=jax.ShapeDtypeStruct((_NC, _NPAD, _D), jnp.float32),
        mesh=_sc_mesh(),
        scratch_types=[
            pltpu.VMEM((_NCHUNK, _CHUNK), jnp.int32),
            pltpu.VMEM((_NCHUNK, _CHUNK), jnp.int32),
            pltpu.VMEM((_EPT_PAD,), jnp.float32),
            pltpu.VMEM((_CHUNK, _D), jnp.float32),
            pltpu.VMEM_SHARED((_NPAD, _D), jnp.float32),
            pltpu.SemaphoreType.DMA,
        ],
        compiler_params=_SC_PARAMS,
        name="sc_agg",
    )
    return kfn(y, srcp, dstp, ewp)


_SQRT_HALF = 1.0 / math.sqrt(2.0)


def _ln(t, w, b):
    mu = jnp.mean(t, axis=-1, keepdims=True)
    var = jnp.mean((t - mu) ** 2, axis=-1, keepdims=True)
    return (t - mu) * lax.rsqrt(var + 1e-5) * w + b


def _gelu(t):
    return 0.5 * t * (1.0 + lax.erf(t * _SQRT_HALF))


def _rows_spec():
    return pl.BlockSpec((_BLK, _D), lambda i: (i, 0))


def _full_spec(shape):
    nd = len(shape)
    return pl.BlockSpec(shape, lambda i: (0,) * nd)


def _prep_body(x_ref, rw_ref, rb_ref, rlw_ref, rlb_ref, w0_ref, id_ref, xw0_ref):
    xb = x_ref[...]
    t = jnp.dot(xb, rw_ref[...], preferred_element_type=jnp.float32) + rb_ref[...]
    id_ref[...] = _gelu(_ln(t, rlw_ref[...], rlb_ref[...]))
    xw0_ref[...] = jnp.dot(xb, w0_ref[...], preferred_element_type=jnp.float32)


def _prep_call(x, res_W, res_b, res_ln_w, res_ln_b, W0):
    return pl.pallas_call(
        _prep_body,
        grid=(_GRID,),
        in_specs=[
            _rows_spec(),
            _full_spec((_D, _D)),
            _full_spec((1, _D)),
            _full_spec((1, _D)),
            _full_spec((1, _D)),
            _full_spec((_D, _D)),
        ],
        out_specs=[_rows_spec(), _rows_spec()],
        out_shape=[
            jax.ShapeDtypeStruct((_N, _D), jnp.float32),
            jax.ShapeDtypeStruct((_N, _D), jnp.float32),
        ],
    )(x, res_W, res_b, res_ln_w, res_ln_b, W0)


def _degp_spec():
    return pl.BlockSpec((_BLK, _NC), lambda i: (i, 0))


def _y0_body(degp_ref, xw_ref, y_ref):
    deg = degp_ref[:, 0] + degp_ref[:, 1] + 1.0
    y_ref[...] = lax.rsqrt(deg)[:, None] * xw_ref[...]


def _y0_call(degp_t, xw0):
    return pl.pallas_call(
        _y0_body,
        grid=(_GRID,),
        in_specs=[_degp_spec(), _rows_spec()],
        out_specs=_rows_spec(),
        out_shape=jax.ShapeDtypeStruct((_N, _D), jnp.float32),
    )(degp_t, xw0)


def _sp_spec():
    return pl.BlockSpec((_NC, _BLK, _D), lambda i: (0, i, 0))


def _post(sp_ref, degp_ref, xw_ref, b_ref, lnw_ref, lnb_ref):
    deg = degp_ref[:, 0] + degp_ref[:, 1] + 1.0
    dis = lax.rsqrt(deg)[:, None]
    s = sp_ref[0] + sp_ref[1]
    agg = dis * s + xw_ref[...] / deg[:, None] + b_ref[...]
    return _gelu(_ln(agg, lnw_ref[...], lnb_ref[...])), dis


def _mid_body(sp_ref, degp_ref, xw_ref, b_ref, lnw_ref, lnb_ref, w_ref,
              xwn_ref, yn_ref):
    o, dis = _post(sp_ref, degp_ref, xw_ref, b_ref, lnw_ref, lnb_ref)
    xwn = jnp.dot(o, w_ref[...], preferred_element_type=jnp.float32)
    xwn_ref[...] = xwn
    yn_ref[...] = dis * xwn


def _mid_call(sp, degp_t, xw, b, lnw, lnb, W_next):
    return pl.pallas_call(
        _mid_body,
        grid=(_GRID,),
        in_specs=[
            _sp_spec(),
            _degp_spec(),
            _rows_spec(),
            _full_spec((1, _D)),
            _full_spec((1, _D)),
            _full_spec((1, _D)),
            _full_spec((_D, _D)),
        ],
        out_specs=[_rows_spec(), _rows_spec()],
        out_shape=[
            jax.ShapeDtypeStruct((_N, _D), jnp.float32),
            jax.ShapeDtypeStruct((_N, _D), jnp.float32),
        ],
    )(sp, degp_t, xw, b, lnw, lnb, W_next)


def _fin_body(sp_ref, degp_ref, xw_ref, b_ref, lnw_ref, lnb_ref, id_ref, out_ref):
    o, _ = _post(sp_ref, degp_ref, xw_ref, b_ref, lnw_ref, lnb_ref)
    out_ref[...] = o + id_ref[...]


def _fin_call(sp, degp_t, xw, b, lnw, lnb, identity):
    return pl.pallas_call(
        _fin_body,
        grid=(_GRID,),
        in_specs=[
            _sp_spec(),
            _degp_spec(),
            _rows_spec(),
            _full_spec((1, _D)),
            _full_spec((1, _D)),
            _full_spec((1, _D)),
            _rows_spec(),
        ],
        out_specs=_rows_spec(),
        out_shape=jax.ShapeDtypeStruct((_N, _D), jnp.float32),
    )(sp, degp_t, xw, b, lnw, lnb, identity)


def kernel(x, edge_index, edge_weight, res_W, res_b, res_ln_w, res_ln_b,
           conv_W, conv_b, ln_w, ln_b):
    x = _f32(x)
    ew = _f32(edge_weight)
    src = jnp.asarray(edge_index[0], jnp.int32)
    dst = jnp.asarray(edge_index[1], jnp.int32)

    pad = _NW * _EPT_PAD - _E
    srcp = jnp.concatenate([src, jnp.zeros((pad,), jnp.int32)])
    dstp = jnp.concatenate([dst, jnp.zeros((pad,), jnp.int32)])
    ewp = jnp.concatenate([ew, jnp.zeros((pad,), jnp.float32)])
    srcp = srcp.reshape(_NW, _NCHUNK, _CHUNK)
    dstp = dstp.reshape(_NW, _NCHUNK, _CHUNK)
    ewf = ewp.reshape(_NW, _EPT_PAD)
    ewp = ewp.reshape(_NW, _NCHUNK, _CHUNK)

    degp = _deg_call(dstp, ewp)
    degp_t = jnp.transpose(degp[:, :_N])

    b2 = lambda v: _f32(v).reshape(1, _D)
    identity, xw = _prep_call(x, _f32(res_W), b2(res_b), b2(res_ln_w),
                              b2(res_ln_b), _f32(conv_W[0]))
    y = _y0_call(degp_t, xw)

    for i in range(_CONVS):
        sp = _agg_call(y, srcp, dstp, ewf)
        if i + 1 < _CONVS:
            xw, y = _mid_call(sp, degp_t, xw, b2(conv_b[i]), b2(ln_w[i]),
                              b2(ln_b[i]), _f32(conv_W[i + 1]))
        else:
            out = _fin_call(sp, degp_t, xw, b2(conv_b[i]), b2(ln_w[i]),
                            b2(ln_b[i]), identity)
    return out
